# Initial kernel scaffold; baseline (speedup 1.0000x reference)
import jax
import jax.numpy as jnp
from jax.experimental import pallas as pl


def kernel(x, table):
    raise NotImplementedError("write your pallas kernel here")


# speedup vs baseline: 4.2122x; 4.2122x over previous
import functools

import jax
import jax.numpy as jnp
from jax import lax
from jax.experimental import pallas as pl
from jax.experimental.pallas import tpu as pltpu
from jax.experimental.pallas import tpu_sc as plsc

VOCAB_PAD = 1024
NC = 2
NS = 16
NW = NC * NS
LANES = 16


def _mesh():
    return plsc.VectorSubcoreMesh(
        core_axis_name="c", subcore_axis_name="s",
        num_cores=NC, num_subcores=NS)


def _worker_id():
    return lax.axis_index("s") * NC + lax.axis_index("c")


@functools.lru_cache(maxsize=None)
def _hist_kernel(bw: int):
    @functools.partial(
        pl.kernel,
        out_type=jax.ShapeDtypeStruct((NW, VOCAB_PAD), jnp.float32),
        mesh=_mesh(),
        scratch_types=[
            pltpu.VMEM((bw,), jnp.int32),
            pltpu.VMEM((VOCAB_PAD,), jnp.float32),
        ],
        compiler_params=pltpu.CompilerParams(needs_layout_passes=False),
    )
    def hist(idx_hbm, out_hbm, idx_v, cnt_v):
        wid = _worker_id()
        pltpu.sync_copy(idx_hbm.at[pl.ds(wid * bw, bw)], idx_v)

        def zero_body(i, carry):
            cnt_v[pl.ds(i * LANES, LANES)] = jnp.zeros((LANES,), jnp.float32)
            return carry
        lax.fori_loop(0, VOCAB_PAD // LANES, zero_body, 0)

        ones = jnp.ones((LANES,), jnp.float32)

        def body(i, carry):
            iv = idx_v[pl.ds(i * LANES, LANES)]
            plsc.addupdate_scatter(cnt_v, [iv], ones)
            return carry
        lax.fori_loop(0, bw // LANES, body, 0)

        pltpu.sync_copy(cnt_v, out_hbm.at[wid])

    return hist


@functools.lru_cache(maxsize=None)
def _norm_kernel(d: int, n_elems: float):
    def body(cnt_ref, tab_t_ref, tab_ref, out_ref):
        cnt = jnp.sum(cnt_ref[...], axis=0, keepdims=True)
        tab_t = tab_t_ref[...]
        row_sum = jnp.sum(tab_t, axis=0, keepdims=True)
        row_sumsq = jnp.sum(tab_t * tab_t, axis=0, keepdims=True)
        s = jnp.sum(cnt * row_sum)
        q = jnp.sum(cnt * row_sumsq)
        mean = s / n_elems
        var = (q - s * mean) / (n_elems - 1.0)
        scale = lax.rsqrt(var) * (1.0 / 6.0)
        out_ref[...] = jnp.clip(
            (tab_ref[...] - mean) * scale + 0.5, 0.0, 1.0)

    return pl.pallas_call(
        body,
        out_shape=jax.ShapeDtypeStruct((VOCAB_PAD, d), jnp.float32),
    )


@functools.lru_cache(maxsize=None)
def _gather_kernel(bt: int, bw: int, d: int):
    SUB = 4
    K = SUB * 128
    nch = bw // K

    @functools.partial(
        pl.kernel,
        out_type=jax.ShapeDtypeStruct((bt, d), jnp.float32),
        mesh=_mesh(),
        scratch_types=[
            pltpu.VMEM((SUB, 128), jnp.int32),
            pltpu.VMEM((K, d), jnp.float32),
            pltpu.SemaphoreType.DMA,
        ],
        compiler_params=pltpu.CompilerParams(
            needs_layout_passes=False, use_tc_tiling_on_sc=False),
    )
    def gather(ntab_hbm, idx_hbm, out_hbm, idx_v, rows_v, gsem):
        wid = _worker_id()
        gbase = wid * (bw // 128)
        rbase = wid * bw

        def chunk(c, carry):
            pltpu.sync_copy(idx_hbm.at[pl.ds(gbase + c * SUB, SUB)], idx_v)
            handles = [
                pltpu.async_copy(
                    ntab_hbm.at[idx_v.at[j]],
                    rows_v.at[pl.ds(j * 128, 128)], gsem)
                for j in range(SUB)
            ]
            for h in handles:
                h.wait()
            pltpu.sync_copy(rows_v, out_hbm.at[pl.ds(rbase + c * K, K)])
            return carry

        lax.fori_loop(0, nch, chunk, 0)

    return gather


def kernel(x, table):
    b, h = x.shape
    v, d = table.shape
    bt = b * h
    bw = bt // NW
    idx_flat = x.reshape(bt).astype(jnp.int32)
    idx_2d = idx_flat.reshape(bt // 128, 128)
    tab_pad = jnp.pad(table, ((0, VOCAB_PAD - v), (0, 0)))
    counts = _hist_kernel(bw)(idx_flat)
    ntab = _norm_kernel(d, float(bt) * d)(counts, tab_pad.T, tab_pad)
    out = _gather_kernel(bt, bw, d)(ntab, idx_2d)
    return out.reshape(b, h, d)


# speedup vs baseline: 4.3360x; 1.0294x over previous
import functools

import jax
import jax.numpy as jnp
from jax import lax
from jax.experimental import pallas as pl
from jax.experimental.pallas import tpu as pltpu
from jax.experimental.pallas import tpu_sc as plsc

VOCAB_PAD = 1024
NC = 2
NS = 16
NW = NC * NS
LANES = 16


def _mesh():
    return plsc.VectorSubcoreMesh(
        core_axis_name="c", subcore_axis_name="s",
        num_cores=NC, num_subcores=NS)


def _worker_id():
    return lax.axis_index("s") * NC + lax.axis_index("c")


@functools.lru_cache(maxsize=None)
def _hist_kernel(bw: int):
    @functools.partial(
        pl.kernel,
        out_type=jax.ShapeDtypeStruct((NW, VOCAB_PAD), jnp.float32),
        mesh=_mesh(),
        scratch_types=[
            pltpu.VMEM((bw,), jnp.int32),
            pltpu.VMEM((VOCAB_PAD,), jnp.float32),
        ],
        compiler_params=pltpu.CompilerParams(needs_layout_passes=False),
    )
    def hist(idx_hbm, out_hbm, idx_v, cnt_v):
        wid = _worker_id()
        pltpu.sync_copy(idx_hbm.at[pl.ds(wid * bw, bw)], idx_v)

        def zero_body(i, carry):
            cnt_v[pl.ds(i * LANES, LANES)] = jnp.zeros((LANES,), jnp.float32)
            return carry
        lax.fori_loop(0, VOCAB_PAD // LANES, zero_body, 0)

        ones = jnp.ones((LANES,), jnp.float32)

        def body(i, carry):
            iv = idx_v[pl.ds(i * LANES, LANES)]
            plsc.addupdate_scatter(cnt_v, [iv], ones)
            return carry
        lax.fori_loop(0, bw // LANES, body, 0)

        pltpu.sync_copy(cnt_v, out_hbm.at[wid])

    return hist


@functools.lru_cache(maxsize=None)
def _norm_kernel(d: int, n_elems: float):
    def body(cnt_ref, tab_t_ref, tab_ref, out_ref):
        cnt = jnp.sum(cnt_ref[...], axis=0, keepdims=True)
        tab_t = tab_t_ref[...]
        row_sum = jnp.sum(tab_t, axis=0, keepdims=True)
        row_sumsq = jnp.sum(tab_t * tab_t, axis=0, keepdims=True)
        s = jnp.sum(cnt * row_sum)
        q = jnp.sum(cnt * row_sumsq)
        mean = s / n_elems
        var = (q - s * mean) / (n_elems - 1.0)
        scale = lax.rsqrt(var) * (1.0 / 6.0)
        out_ref[...] = jnp.clip(
            (tab_ref[...] - mean) * scale + 0.5, 0.0, 1.0)

    return pl.pallas_call(
        body,
        out_shape=jax.ShapeDtypeStruct((VOCAB_PAD, d), jnp.float32),
    )


@functools.lru_cache(maxsize=None)
def _gather_kernel(nb: int, nh: int, d: int):
    GW = 100
    CB = 4
    kk = CB * nh
    ng = kk // GW
    bwb = nb // NW
    nch = bwb // CB
    NBUF = 2

    @functools.partial(
        pl.kernel,
        out_type=jax.ShapeDtypeStruct((nb, nh, d), jnp.float32),
        mesh=_mesh(),
        scratch_types=[
            pltpu.VMEM((NBUF, ng, GW), jnp.int32),
            pltpu.VMEM((NBUF, CB, nh, d), jnp.float32),
            pltpu.SemaphoreType.DMA,
            pltpu.SemaphoreType.DMA,
            pltpu.SemaphoreType.DMA,
        ],
        compiler_params=pltpu.CompilerParams(
            needs_layout_passes=False, use_tc_tiling_on_sc=False),
    )
    def gather(ntab_hbm, idx_hbm, out_hbm, idx_v, rows_v, isem, gsem, osem):
        wid = _worker_id()
        gbase = wid * (bwb * nh // GW)
        bbase = wid * bwb

        def idx_copy(c, b):
            return pltpu.async_copy(
                idx_hbm.at[pl.ds(gbase + c * ng, ng)], idx_v.at[b], isem)

        def fire_gathers(b):
            for j in range(ng):
                pltpu.async_copy(
                    ntab_hbm.at[idx_v.at[b].at[j]],
                    rows_v.at[b].at[j // 2, pl.ds((j % 2) * GW, GW)],
                    gsem)

        def wait_gathers(b):
            for j in range(ng):
                pltpu.make_async_copy(
                    ntab_hbm.at[idx_v.at[b].at[j]],
                    rows_v.at[b].at[j // 2, pl.ds((j % 2) * GW, GW)],
                    gsem).wait()

        def out_store(c, b):
            return pltpu.async_copy(
                rows_v.at[b], out_hbm.at[pl.ds(bbase + c * CB, CB)], osem)

        def wait_out_store(c, b):
            pltpu.make_async_copy(
                rows_v.at[b], out_hbm.at[pl.ds(bbase + c * CB, CB)],
                osem).wait()

        idx_copy(0, 0).wait()
        fire_gathers(0)
        idx_copy(1, 1)

        def pipe(c2, carry):
            for b in range(NBUF):
                other = 1 - b
                c = c2 * NBUF + b
                wait_gathers(b)
                out_store(c, b)

                @pl.when(c + 1 < nch)
                def _():
                    pltpu.make_async_copy(
                        idx_hbm.at[pl.ds(gbase + (c + 1) * ng, ng)],
                        idx_v.at[other], isem).wait()

                    @pl.when(c >= 1)
                    def _():
                        wait_out_store(c - 1, other)
                    fire_gathers(other)

                    @pl.when(c + 2 < nch)
                    def _():
                        idx_copy(c + 2, b)
            return carry

        lax.fori_loop(0, nch // NBUF, pipe, 0)
        wait_out_store(nch - 1, (nch - 1) % NBUF)

    return gather


def kernel(x, table):
    nb, nh = x.shape
    v, d = table.shape
    bt = nb * nh
    bw = bt // NW
    idx_flat = x.reshape(bt).astype(jnp.int32)
    idx_2d = idx_flat.reshape(bt // 100, 100)
    tab_pad = jnp.pad(table, ((0, VOCAB_PAD - v), (0, 0)))
    counts = _hist_kernel(bw)(idx_flat)
    ntab = _norm_kernel(d, float(bt) * d)(counts, tab_pad.T, tab_pad)
    return _gather_kernel(nb, nh, d)(ntab, idx_2d)
